# baseline (speedup 1.0000x reference)
import functools

import jax
import jax.numpy as jnp
from jax import lax
from jax.experimental import pallas as pl
from jax.experimental.pallas import tpu as pltpu
from jax.experimental.pallas import tpu_sc as plsc

MAXLEN = 200
EMBED = 32
BATCH = 4096
VOCAB = 1000000

NC = 2
NSUB = 16
NW = NC * NSUB
BW = BATCH // NW
LANES = 16
DEPTH = 8


NBLK = VOCAB // 128
NTAIL = VOCAB - NBLK * 128
NBLKW = (NBLK + 31) // 32


def _make_detile_kernel():
    mesh = plsc.VectorSubcoreMesh(core_axis_name="c", subcore_axis_name="s")

    @functools.partial(
        pl.kernel,
        mesh=mesh,
        compiler_params=pltpu.CompilerParams(
            use_tc_tiling_on_sc=True, needs_layout_passes=False),
        out_type=jax.ShapeDtypeStruct((VOCAB // 4, 128), jnp.float32),
        scratch_types=(
            [pltpu.VMEM((EMBED // 8, 8, 128), jnp.float32)] * 2
            + [pltpu.VMEM((EMBED // 8, 8, 128), jnp.float32)] * 2
            + [pltpu.SemaphoreType.DMA] * 2
            + [pltpu.SemaphoreType.DMA] * 2
        ),
    )
    def detile_kernel(tokT_hbm, tail_hbm, out_hbm, ib0, ib1, ob0, ob1,
                      gs0, gs1, os0, os1):
        wid = lax.axis_index("s") * NC + lax.axis_index("c")
        IB, OB, GS, OS = (ib0, ib1), (ob0, ob1), (gs0, gs1), (os0, os1)

        iota = jnp.arange(LANES, dtype=jnp.int32)
        tok_vecs = [iota + t0 * LANES for t0 in range(8)]
        rr_vecs = [(tv >> 2) for tv in tok_vecs]
        ti_out = [rv >> 3 for rv in rr_vecs]
        row_out = [rv & 7 for rv in rr_vecs]
        l_base = [(tv & 3) * EMBED for tv in tok_vecs]

        def fire_in(c, ib, sem):
            for dh in range(EMBED // 8):
                pltpu.async_copy(
                    tokT_hbm.at[pl.ds(dh * 8, 8), pl.ds(c * 128, 128)],
                    ib.at[dh], sem)

        def drain_in(ib, sem):
            for dh in range(EMBED // 8):
                pltpu.make_async_copy(
                    tokT_hbm.at[pl.ds(0, 8), pl.ds(0, 128)],
                    ib.at[dh], sem).wait()

        def permute(ib, ob):
            @plsc.parallel_loop(0, EMBED, 1, unroll=2)
            def _(d0):
                dmod = (iota + d0) & (EMBED - 1)
                ti_in = dmod >> 3
                row_in = dmod & 7
                for t0 in range(8):
                    v = plsc.load_gather(ib, [ti_in, row_in, tok_vecs[t0]])
                    plsc.store_scatter(
                        ob, [ti_out[t0], row_out[t0], l_base[t0] + dmod], v)

        def fire_out(c, ob, sem):
            for ro in range(EMBED // 8):
                pltpu.async_copy(
                    ob.at[ro], out_hbm.at[pl.ds(c * 32 + ro * 8, 8), :], sem)

        def drain_out(ob, sem):
            for ro in range(EMBED // 8):
                pltpu.make_async_copy(
                    ob.at[ro], out_hbm.at[pl.ds(0, 8), :], sem).wait()

        nblk = (NBLK - 1 - wid) // NW + 1
        last = (nblk - 1) * NW + wid

        def col(k):
            return jnp.minimum(wid + k * NW, last)

        def step(k, g, p):
            drain_in(IB[p], GS[p])

            @pl.when(g >= 1)
            def _():
                drain_out(OB[p], OS[p])

            permute(IB[p], OB[p])
            fire_out(col(k), OB[p], OS[p])

        fire_in(col(0), IB[0], GS[0])

        def body(g, _):
            k0 = 2 * g
            fire_in(col(k0 + 1), IB[1], GS[1])
            step(k0, g, 0)
            fire_in(col(k0 + 2), IB[0], GS[0])
            step(k0 + 1, g, 1)
            return _

        lax.fori_loop(0, (NBLKW - 1) // 2, body, None)

        drain_in(IB[0], GS[0])
        drain_out(OB[0], OS[0])
        permute(IB[0], OB[0])
        fire_out(col(NBLKW - 1), OB[0], OS[0])
        drain_out(OB[1], OS[1])
        drain_out(OB[0], OS[0])

        @pl.when(wid == NW - 1)
        def _():
            for ro in range(NTAIL // 32):
                pltpu.sync_copy(tail_hbm.at[pl.ds(ro * 8, 8), :],
                                IB[0].at[ro])
                pltpu.sync_copy(IB[0].at[ro],
                                out_hbm.at[pl.ds(NBLK * 32 + ro * 8, 8), :])

    return detile_kernel


def _make_sc_kernel():
    mesh = plsc.VectorSubcoreMesh(core_axis_name="c", subcore_axis_name="s")

    @functools.partial(
        pl.kernel,
        mesh=mesh,
        compiler_params=pltpu.CompilerParams(
            use_tc_tiling_on_sc=False, needs_layout_passes=False),
        out_type=jax.ShapeDtypeStruct((MAXLEN, EMBED // 8, NW, 8, BW), jnp.float32),
        scratch_types=(
            [
                pltpu.VMEM((MAXLEN, BW), jnp.int32),
                pltpu.VMEM((MAXLEN, EMBED), jnp.float32),
            ]
            + [pltpu.VMEM((BW, EMBED), jnp.float32)] * DEPTH
            + [pltpu.VMEM((EMBED, BW + 1), jnp.float32)] * DEPTH
            + [pltpu.SemaphoreType.DMA] * DEPTH
            + [pltpu.SemaphoreType.DMA] * DEPTH
        ),
    )
    def emb_kernel(idxT_hbm, tok_hbm, pos_hbm, out_hbm, idx_v, pos_v, *bufs):
        RB = bufs[:DEPTH]
        TB = bufs[DEPTH:2 * DEPTH]
        GS = bufs[2 * DEPTH:3 * DEPTH]
        OS = bufs[3 * DEPTH:4 * DEPTH]
        wid = lax.axis_index("s") * NC + lax.axis_index("c")

        pltpu.sync_copy(idxT_hbm.at[:, pl.ds(wid * BW, BW)], idx_v)
        pltpu.sync_copy(pos_hbm, pos_v)

        iota = jnp.arange(LANES, dtype=jnp.int32)
        dv0 = iota
        dv1 = iota + LANES

        def fire(t, rb, sem):
            pltpu.async_copy(tok_hbm.at[idx_v.at[t]], rb, sem)

        def drain_g(rb, sem):
            pltpu.make_async_copy(tok_hbm.at[pl.ds(0, BW)], rb, sem).wait()

        def trans_add(t, rb, tb):
            pv0 = pos_v[t, pl.ds(0, LANES)]
            pv1 = pos_v[t, pl.ds(LANES, LANES)]

            @plsc.parallel_loop(0, BW, 1, unroll=8)
            def _(b):
                bv = jnp.full((LANES,), b, dtype=jnp.int32)
                v0 = rb[b, pl.ds(0, LANES)] + pv0
                plsc.store_scatter(tb, [dv0, bv], v0)
                v1 = rb[b, pl.ds(LANES, LANES)] + pv1
                plsc.store_scatter(tb, [dv1, bv], v1)

        def out_fire(t, tb, sem):
            for dh in range(EMBED // 8):
                pltpu.async_copy(
                    tb.at[pl.ds(dh * 8, 8), pl.ds(0, BW)],
                    out_hbm.at[t, dh, wid], sem)

        def out_drain(tb, sem):
            for dh in range(EMBED // 8):
                pltpu.make_async_copy(
                    tb.at[pl.ds(dh * 8, 8), pl.ds(0, BW)],
                    out_hbm.at[0, dh, wid], sem).wait()

        for p in range(DEPTH):
            fire(p, RB[p], GS[p])
        for p in range(DEPTH):
            drain_g(RB[p], GS[p])
            trans_add(p, RB[p], TB[p])
            fire(p + DEPTH, RB[p], GS[p])
            out_fire(p, TB[p], OS[p])

        def body(gg, _):
            t0 = DEPTH * gg + DEPTH
            for p in range(DEPTH):
                out_drain(TB[p], OS[p])
                drain_g(RB[p], GS[p])
                trans_add(t0 + p, RB[p], TB[p])
                fire(t0 + p + DEPTH, RB[p], GS[p])
                out_fire(t0 + p, TB[p], OS[p])
            return _

        lax.fori_loop(0, MAXLEN // DEPTH - 2, body, None)

        for p in range(DEPTH):
            t = MAXLEN - DEPTH + p
            out_drain(TB[p], OS[p])
            drain_g(RB[p], GS[p])
            trans_add(t, RB[p], TB[p])
            out_fire(t, TB[p], OS[p])
        for p in range(DEPTH):
            out_drain(TB[p], OS[p])

    return emb_kernel


_EMB_KERNEL = _make_sc_kernel()
_DETILE = _make_detile_kernel()


def kernel(inputs, token_table, pos_table):
    idx_t = inputs.astype(jnp.int32).T
    tail16 = token_table[NBLK * 128:].reshape(NTAIL // 4, 128)
    table_lin = _DETILE(token_table.T, tail16).reshape(VOCAB, EMBED)
    z = _EMB_KERNEL(idx_t, table_lin, pos_table)
    return z.transpose(2, 4, 0, 1, 3).reshape(BATCH, MAXLEN, EMBED)


# speedup vs baseline: 1.0099x; 1.0099x over previous
import functools

import jax
import jax.numpy as jnp
from jax import lax
from jax.experimental import pallas as pl
from jax.experimental.pallas import tpu as pltpu
from jax.experimental.pallas import tpu_sc as plsc

MAXLEN = 200
EMBED = 32
BATCH = 4096
VOCAB = 1000000

NC = 2
NSUB = 16
NW = NC * NSUB
BW = BATCH // NW
LANES = 16
DEPTH = 5


NBLK = VOCAB // 128
NTAIL = VOCAB - NBLK * 128
NBLKW = (NBLK + 31) // 32


def _make_detile_kernel():
    mesh = plsc.VectorSubcoreMesh(core_axis_name="c", subcore_axis_name="s")

    @functools.partial(
        pl.kernel,
        mesh=mesh,
        compiler_params=pltpu.CompilerParams(
            use_tc_tiling_on_sc=True, needs_layout_passes=False),
        out_type=jax.ShapeDtypeStruct((VOCAB // 4, 128), jnp.float32),
        scratch_types=(
            [pltpu.VMEM((EMBED // 8, 8, 128), jnp.float32)] * 2
            + [pltpu.VMEM((EMBED // 8, 8, 128), jnp.float32)] * 2
            + [pltpu.SemaphoreType.DMA] * 2
            + [pltpu.SemaphoreType.DMA] * 2
        ),
    )
    def detile_kernel(tokT_hbm, tail_hbm, out_hbm, ib0, ib1, ob0, ob1,
                      gs0, gs1, os0, os1):
        wid = lax.axis_index("s") * NC + lax.axis_index("c")
        IB, OB, GS, OS = (ib0, ib1), (ob0, ob1), (gs0, gs1), (os0, os1)

        iota = jnp.arange(LANES, dtype=jnp.int32)
        tok_vecs = [iota + t0 * LANES for t0 in range(8)]
        rr_vecs = [(tv >> 2) for tv in tok_vecs]
        ti_out = [rv >> 3 for rv in rr_vecs]
        row_out = [rv & 7 for rv in rr_vecs]
        l_base = [(tv & 3) * EMBED for tv in tok_vecs]

        def fire_in(c, ib, sem):
            for dh in range(EMBED // 8):
                pltpu.async_copy(
                    tokT_hbm.at[pl.ds(dh * 8, 8), pl.ds(c * 128, 128)],
                    ib.at[dh], sem)

        def drain_in(ib, sem):
            for dh in range(EMBED // 8):
                pltpu.make_async_copy(
                    tokT_hbm.at[pl.ds(0, 8), pl.ds(0, 128)],
                    ib.at[dh], sem).wait()

        def permute(ib, ob):
            @plsc.parallel_loop(0, EMBED, 1, unroll=2)
            def _(d0):
                dmod = (iota + d0) & (EMBED - 1)
                ti_in = dmod >> 3
                row_in = dmod & 7
                for t0 in range(8):
                    v = plsc.load_gather(ib, [ti_in, row_in, tok_vecs[t0]])
                    plsc.store_scatter(
                        ob, [ti_out[t0], row_out[t0], l_base[t0] + dmod], v)

        def fire_out(c, ob, sem):
            for ro in range(EMBED // 8):
                pltpu.async_copy(
                    ob.at[ro], out_hbm.at[pl.ds(c * 32 + ro * 8, 8), :], sem)

        def drain_out(ob, sem):
            for ro in range(EMBED // 8):
                pltpu.make_async_copy(
                    ob.at[ro], out_hbm.at[pl.ds(0, 8), :], sem).wait()

        nblk = (NBLK - 1 - wid) // NW + 1
        last = (nblk - 1) * NW + wid

        def col(k):
            return jnp.minimum(wid + k * NW, last)

        def step(k, g, p):
            drain_in(IB[p], GS[p])

            @pl.when(g >= 1)
            def _():
                drain_out(OB[p], OS[p])

            permute(IB[p], OB[p])
            fire_out(col(k), OB[p], OS[p])

        fire_in(col(0), IB[0], GS[0])

        def body(g, _):
            k0 = 2 * g
            fire_in(col(k0 + 1), IB[1], GS[1])
            step(k0, g, 0)
            fire_in(col(k0 + 2), IB[0], GS[0])
            step(k0 + 1, g, 1)
            return _

        lax.fori_loop(0, (NBLKW - 1) // 2, body, None)

        drain_in(IB[0], GS[0])
        drain_out(OB[0], OS[0])
        permute(IB[0], OB[0])
        fire_out(col(NBLKW - 1), OB[0], OS[0])
        drain_out(OB[1], OS[1])
        drain_out(OB[0], OS[0])

        @pl.when(wid == NW - 1)
        def _():
            for ro in range(NTAIL // 32):
                pltpu.sync_copy(tail_hbm.at[pl.ds(ro * 8, 8), :],
                                IB[0].at[ro])
                pltpu.sync_copy(IB[0].at[ro],
                                out_hbm.at[pl.ds(NBLK * 32 + ro * 8, 8), :])

    return detile_kernel


def _make_sc_kernel():
    mesh = plsc.VectorSubcoreMesh(core_axis_name="c", subcore_axis_name="s")

    @functools.partial(
        pl.kernel,
        mesh=mesh,
        compiler_params=pltpu.CompilerParams(
            use_tc_tiling_on_sc=False, needs_layout_passes=False),
        out_type=jax.ShapeDtypeStruct((MAXLEN, EMBED // 8, NW, 8, BW), jnp.float32),
        scratch_types=(
            [
                pltpu.VMEM((MAXLEN, BW), jnp.int32),
                pltpu.VMEM((MAXLEN, EMBED), jnp.float32),
            ]
            + [pltpu.VMEM((BW, EMBED), jnp.float32)] * DEPTH
            + [pltpu.VMEM((EMBED, BW + 1), jnp.float32)] * DEPTH
            + [pltpu.SemaphoreType.DMA] * DEPTH
            + [pltpu.SemaphoreType.DMA] * DEPTH
        ),
    )
    def emb_kernel(idxT_hbm, tok_hbm, pos_hbm, out_hbm, idx_v, pos_v, *bufs):
        RB = bufs[:DEPTH]
        TB = bufs[DEPTH:2 * DEPTH]
        GS = bufs[2 * DEPTH:3 * DEPTH]
        OS = bufs[3 * DEPTH:4 * DEPTH]
        wid = lax.axis_index("s") * NC + lax.axis_index("c")

        pltpu.sync_copy(idxT_hbm.at[:, pl.ds(wid * BW, BW)], idx_v)
        pltpu.sync_copy(pos_hbm, pos_v)

        iota = jnp.arange(LANES, dtype=jnp.int32)
        dv0 = iota
        dv1 = iota + LANES

        def fire(t, rb, sem):
            pltpu.async_copy(tok_hbm.at[idx_v.at[t]], rb, sem)

        def drain_g(rb, sem):
            pltpu.make_async_copy(tok_hbm.at[pl.ds(0, BW)], rb, sem).wait()

        def trans_add(t, rb, tb):
            pv0 = pos_v[t, pl.ds(0, LANES)]
            pv1 = pos_v[t, pl.ds(LANES, LANES)]

            @plsc.parallel_loop(0, BW, 1, unroll=8)
            def _(b):
                bv = jnp.full((LANES,), b, dtype=jnp.int32)
                v0 = rb[b, pl.ds(0, LANES)] + pv0
                plsc.store_scatter(tb, [dv0, bv], v0)
                v1 = rb[b, pl.ds(LANES, LANES)] + pv1
                plsc.store_scatter(tb, [dv1, bv], v1)

        def out_fire(t, tb, sem):
            for dh in range(EMBED // 8):
                pltpu.async_copy(
                    tb.at[pl.ds(dh * 8, 8), pl.ds(0, BW)],
                    out_hbm.at[t, dh, wid], sem)

        def out_drain(tb, sem):
            for dh in range(EMBED // 8):
                pltpu.make_async_copy(
                    tb.at[pl.ds(dh * 8, 8), pl.ds(0, BW)],
                    out_hbm.at[0, dh, wid], sem).wait()

        for p in range(DEPTH):
            fire(p, RB[p], GS[p])
        for p in range(DEPTH):
            drain_g(RB[p], GS[p])
            trans_add(p, RB[p], TB[p])
            fire(p + DEPTH, RB[p], GS[p])
            out_fire(p, TB[p], OS[p])

        def body(gg, _):
            t0 = DEPTH * gg + DEPTH
            for p in range(DEPTH):
                out_drain(TB[p], OS[p])
                drain_g(RB[p], GS[p])
                trans_add(t0 + p, RB[p], TB[p])
                fire(t0 + p + DEPTH, RB[p], GS[p])
                out_fire(t0 + p, TB[p], OS[p])
            return _

        lax.fori_loop(0, MAXLEN // DEPTH - 2, body, None)

        for p in range(DEPTH):
            t = MAXLEN - DEPTH + p
            out_drain(TB[p], OS[p])
            drain_g(RB[p], GS[p])
            trans_add(t, RB[p], TB[p])
            out_fire(t, TB[p], OS[p])
        for p in range(DEPTH):
            out_drain(TB[p], OS[p])

    return emb_kernel


_EMB_KERNEL = _make_sc_kernel()
_DETILE = _make_detile_kernel()


def kernel(inputs, token_table, pos_table):
    idx_t = inputs.astype(jnp.int32).T
    tail16 = token_table[NBLK * 128:].reshape(NTAIL // 4, 128)
    table_lin = _DETILE(token_table.T, tail16).reshape(VOCAB, EMBED)
    z = _EMB_KERNEL(idx_t, table_lin, pos_table)
    return z.transpose(2, 4, 0, 1, 3).reshape(BATCH, MAXLEN, EMBED)
